# baseline (speedup 1.0000x reference)
import functools

import jax
import jax.numpy as jnp
import numpy as np
from jax import lax
from jax.experimental import pallas as pl
from jax.experimental.pallas import tpu as pltpu

ROWS, LANES = 2048, 128
P = ROWS * LANES
LOG2P = 18
NSTAGE = LOG2P * (LOG2P + 1) // 2


FUSE = 9


def _stage_params():
    rows = []
    for kk in range(1, LOG2P + 1):
        K = 1 << kk
        for jj in range(kk - 1, -1, -1):
            J = 1 << jj
            k_row = K >= LANES * 2 or kk == LOG2P
            kbit = (K // LANES) if k_row else K
            if J >= LANES:
                r = J // LANES
                rows.append((0, r, kbit, (ROWS - r) % ROWS, r))
            elif k_row:
                rows.append((1, J, kbit, (LANES - J) % LANES, J))
            else:
                rows.append((2, J, kbit, (LANES - J) % LANES, J))
    return np.asarray(rows, dtype=np.int32)


def _body(params_ref, logits_ref, target_ref, out_ref, x_ref, g_ref):
    b = pl.program_id(0)
    s = pl.program_id(1)

    row = lax.broadcasted_iota(jnp.int32, (ROWS, LANES), 0)
    lane = lax.broadcasted_iota(jnp.int32, (ROWS, LANES), 1)

    @pl.when(s == 0)
    def _pack():
        lab = target_ref[0]
        labf = lab.astype(jnp.float32)
        e = 1.0 - logits_ref[0] * (2.0 * labf - 1.0)
        epos = jnp.maximum(e, 0.0)
        bits = (lax.bitcast_convert_type(epos, jnp.int32) & jnp.int32(~1)) | lab
        x_ref[...] = lax.bitcast_convert_type(bits, jnp.float32)
        g_ref[0] = jnp.sum(labf)

    def _substage(t, x):
        i = s * FUSE + t
        case = params_ref[i, 0]
        jbit = params_ref[i, 1]
        kbit = params_ref[i, 2]
        sh_m = params_ref[i, 3]
        sh_p = params_ref[i, 4]

        def _cx(x, low, desc, axis):
            xm = pltpu.roll(x, sh_m, axis=axis)
            xp_ = pltpu.roll(x, sh_p, axis=axis)
            partner = jnp.where(low, xm, xp_)
            keep_max = low == desc
            return jnp.where(keep_max, jnp.maximum(x, partner),
                             jnp.minimum(x, partner))

        return lax.switch(case, [
            lambda x: _cx(x, (row & jbit) == 0, (row & kbit) == 0, 0),
            lambda x: _cx(x, (lane & jbit) == 0, (row & kbit) == 0, 1),
            lambda x: _cx(x, (lane & jbit) == 0, (lane & kbit) == 0, 1),
        ], x)

    x = x_ref[...]
    for t in range(FUSE):
        x = _substage(t, x)
    x_ref[...] = x

    @pl.when(s == NSTAGE // FUSE - 1)
    def _eval():
        sbits = lax.bitcast_convert_type(x_ref[...], jnp.int32)
        l_sorted = (sbits & 1).astype(jnp.float32)
        e_sorted = lax.bitcast_convert_type(sbits & jnp.int32(~1), jnp.float32)

        cs = l_sorted
        for sh in (1, 2, 4, 8, 16, 32, 64):
            cs = cs + jnp.where(lane >= sh, pltpu.roll(cs, sh, axis=1), 0.0)
        rt = cs[:, LANES - 1:LANES]
        rs = rt
        rowv = lax.broadcasted_iota(jnp.int32, (ROWS, 1), 0)
        for sh in (1, 2, 4, 8, 16, 32, 64, 128, 256, 512, 1024):
            rs = rs + jnp.where(rowv >= sh, pltpu.roll(rs, sh, axis=0), 0.0)
        c = cs + (rs - rt)

        G = g_ref[0]
        k = (row * LANES + lane).astype(jnp.float32) + 1.0
        n = k - c
        cm1 = c - l_sorted
        nm1 = (k - 1.0) - cm1
        jk = k / (G + n)
        jm1 = (k - 1.0) / jnp.maximum(G + nm1, 1.0)
        loss = jnp.sum(e_sorted * (jk - jm1))

        prev = jnp.where(b == 0, 0.0, out_ref[0, 0])
        out_ref[0, 0] = prev + loss * (1.0 / 8.0)


@functools.partial(jax.jit, static_argnames=("interpret",))
def _run(logits, target, interpret=False):
    B = logits.shape[0]
    lg = logits.reshape(B, ROWS, LANES)
    tg = target.reshape(B, ROWS, LANES)
    params = jnp.asarray(_stage_params())

    out = pl.pallas_call(
        _body,
        grid=(B, NSTAGE // FUSE),
        in_specs=[
            pl.BlockSpec(memory_space=pltpu.SMEM),
            pl.BlockSpec((1, ROWS, LANES), lambda b, s: (b, 0, 0)),
            pl.BlockSpec((1, ROWS, LANES), lambda b, s: (b, 0, 0)),
        ],
        out_specs=pl.BlockSpec((1, 1), lambda b, s: (0, 0),
                               memory_space=pltpu.SMEM),
        out_shape=jax.ShapeDtypeStruct((1, 1), jnp.float32),
        scratch_shapes=[
            pltpu.VMEM((ROWS, LANES), jnp.float32),
            pltpu.SMEM((1,), jnp.float32),
        ],
        interpret=interpret,
    )(params, lg, tg)
    return out.reshape(())


def kernel(logits, target):
    return _run(logits, target)


# speedup vs baseline: 5.9880x; 5.9880x over previous
import functools

import jax
import jax.numpy as jnp
import numpy as np
from jax import lax
from jax.experimental import pallas as pl
from jax.experimental.pallas import tpu as pltpu
from jax.experimental.pallas import tpu_sc as plsc

ROWS, LANES = 2048, 128
P = ROWS * LANES
B = 8
NT = 16
CHUNK = P // NT
NV = CHUNK // 16
IMGS_PER_CORE = 4


def _pack_body(logits_ref, target_ref, keys_ref):
    lab = target_ref[0]
    labf = lab.astype(jnp.float32)
    e = 1.0 - logits_ref[0] * (2.0 * labf - 1.0)
    epos = jnp.maximum(e, 0.0)
    bits = (lax.bitcast_convert_type(epos, jnp.int32) & jnp.int32(~1)) | lab
    keys_ref[0] = bits


def _pack(lg, tg):
    return pl.pallas_call(
        _pack_body,
        grid=(B,),
        in_specs=[
            pl.BlockSpec((1, ROWS, LANES), lambda b: (b, 0, 0)),
            pl.BlockSpec((1, ROWS, LANES), lambda b: (b, 0, 0)),
        ],
        out_specs=pl.BlockSpec((1, ROWS, LANES), lambda b: (b, 0, 0)),
        out_shape=jax.ShapeDtypeStruct((B, ROWS, LANES), jnp.int32),
    )(lg, tg)


def _take16(vec, idx):
    return lax.gather(
        vec, idx[:, None],
        dimension_numbers=lax.GatherDimensionNumbers(
            offset_dims=(), collapsed_slice_dims=(0,), start_index_map=(0,)),
        slice_sizes=(1,),
        mode=lax.GatherScatterMode.PROMISE_IN_BOUNDS)


def _sc_sort(keys):
    mesh = plsc.VectorSubcoreMesh(core_axis_name="c", subcore_axis_name="s")

    @functools.partial(
        pl.kernel,
        mesh=mesh,
        compiler_params=pltpu.CompilerParams(use_tc_tiling_on_sc=False,
                                             needs_layout_passes=False),
        out_type=jax.ShapeDtypeStruct((B, P), jnp.int32),
        scratch_types=[
            pltpu.VMEM((CHUNK,), jnp.int32),
            pltpu.VMEM((CHUNK,), jnp.int32),
            pltpu.VMEM((CHUNK,), jnp.int32),
            pltpu.VMEM((CHUNK,), jnp.int32),
            pltpu.VMEM((16, 256), jnp.int32),
            pltpu.VMEM((16, 256), jnp.int32),
            pltpu.VMEM((256,), jnp.int32),
            pltpu.VMEM((256,), jnp.int32),
            pltpu.VMEM((NT, 256), jnp.int32),
            pltpu.VMEM_SHARED((P,), jnp.int32),
            pltpu.VMEM_SHARED((P,), jnp.int32),
            pltpu.VMEM_SHARED((NT, 256), jnp.int32),
            pltpu.SemaphoreType.DMA,
        ],
    )
    def k(keys_hbm, out_hbm, buf, valbuf, posbuf, dbuf, cnt2, pfx2, offs,
          hist, hists_local, S0, S1, hist_sh, sem):
        c = lax.axis_index("c")
        t = lax.axis_index("s")
        lane = lax.iota(jnp.int32, 16)
        zeros16 = jnp.zeros((16,), jnp.int32)
        lane_nv = lane * NV
        my = pl.ds(t * CHUNK, CHUNK)

        for l in range(16):
            for ch in range(16):
                cnt2[l, pl.ds(ch * 16, 16)] = zeros16

        def do_pass(shift, dst, last=False):
            def sweep(q, _):
                sl = pl.ds(q * 16, 16)
                v = buf[sl]
                d = 255 - ((v >> shift) & 255)
                cg = plsc.load_gather(cnt2, [lane, d])
                plsc.store_scatter(cnt2, [lane, d], cg + 1)
                posbuf[sl] = cg
                valbuf[sl] = v
                dbuf[sl] = d
                return 0

            lax.fori_loop(0, NV, sweep, 0)

            for ch in range(16):
                chs = pl.ds(ch * 16, 16)
                acc = zeros16
                for l in range(16):
                    rowv = cnt2[l, chs]
                    pfx2[l, chs] = acc
                    cnt2[l, chs] = zeros16
                    acc = acc + rowv
                hist[chs] = acc

            pltpu.sync_copy(hist, hist_sh.at[t])
            plsc.subcore_barrier()
            pltpu.sync_copy(hist_sh, hists_local)

            carry = jnp.int32(0)
            for ch in range(16):
                col = zeros16
                part = zeros16
                for tt in range(16):
                    h = hists_local[tt, pl.ds(ch * 16, 16)]
                    col = col + h
                    tv = jnp.full((16,), tt, jnp.int32)
                    part = part + jnp.where(tv < t, h, zeros16)
                incl = plsc.cumsum(col)
                ov = (incl - col) + carry + part
                carry = carry + jnp.sum(col)
                chs = pl.ds(ch * 16, 16)
                for l in range(16):
                    pfx2[l, chs] = pfx2[l, chs] + ov

            def fix(q, _):
                sl = pl.ds(q * 16, 16)
                dv = dbuf[sl]
                pos = posbuf[sl] + plsc.load_gather(pfx2, [lane, dv])
                if not last:
                    pos = ((pos & ~jnp.int32(16383)) | ((pos & 1023) << 4)
                           | ((pos >> 10) & 15))
                posbuf[sl] = pos
                return 0

            lax.fori_loop(0, NV, fix, 0)
            pltpu.async_copy(valbuf, dst.at[posbuf], sem).wait()
            plsc.subcore_barrier()

        def img_body(ii, _):
            img = c * IMGS_PER_CORE + ii
            pltpu.sync_copy(keys_hbm.at[img, my], buf)
            do_pass(0, S0)
            pltpu.sync_copy(S0.at[my], buf)
            do_pass(8, S1)
            pltpu.sync_copy(S1.at[my], buf)
            do_pass(16, S0)
            pltpu.sync_copy(S0.at[my], buf)
            do_pass(24, S1, last=True)
            pltpu.sync_copy(S1.at[my], out_hbm.at[img, my])
            plsc.subcore_barrier()
            return 0

        lax.fori_loop(0, IMGS_PER_CORE, img_body, 0)

    return k(keys)


def _eval_body(skeys_ref, out_ref):
    b = pl.program_id(0)
    row = lax.broadcasted_iota(jnp.int32, (ROWS, LANES), 0)
    lane = lax.broadcasted_iota(jnp.int32, (ROWS, LANES), 1)

    sbits = skeys_ref[0]
    l_sorted = (sbits & 1).astype(jnp.float32)
    e_sorted = lax.bitcast_convert_type(sbits & jnp.int32(~1), jnp.float32)
    G = jnp.sum(l_sorted)

    cs = l_sorted
    for sh in (1, 2, 4, 8, 16, 32, 64):
        cs = cs + jnp.where(lane >= sh, pltpu.roll(cs, sh, axis=1), 0.0)
    rt = cs[:, LANES - 1:LANES]
    rs = rt
    rowv = lax.broadcasted_iota(jnp.int32, (ROWS, 1), 0)
    for sh in (1, 2, 4, 8, 16, 32, 64, 128, 256, 512, 1024):
        rs = rs + jnp.where(rowv >= sh, pltpu.roll(rs, sh, axis=0), 0.0)
    c = cs + (rs - rt)

    k = (row * LANES + lane).astype(jnp.float32) + 1.0
    n = k - c
    cm1 = c - l_sorted
    nm1 = (k - 1.0) - cm1
    jk = k / (G + n)
    jm1 = (k - 1.0) / jnp.maximum(G + nm1, 1.0)
    loss = jnp.sum(e_sorted * (jk - jm1))

    prev = jnp.where(b == 0, 0.0, out_ref[0, 0])
    out_ref[0, 0] = prev + loss * (1.0 / B)


def _eval(skeys):
    out = pl.pallas_call(
        _eval_body,
        grid=(B,),
        in_specs=[pl.BlockSpec((1, ROWS, LANES), lambda b: (b, 0, 0))],
        out_specs=pl.BlockSpec((1, 1), lambda b: (0, 0),
                               memory_space=pltpu.SMEM),
        out_shape=jax.ShapeDtypeStruct((1, 1), jnp.float32),
    )(skeys)
    return out.reshape(())


@jax.jit
def _run(logits, target):
    lg = logits.reshape(B, ROWS, LANES)
    tg = target.reshape(B, ROWS, LANES)
    keys = _pack(lg, tg).reshape(B, P)
    skeys = _sc_sort(keys)
    return _eval(skeys.reshape(B, ROWS, LANES))


def kernel(logits, target):
    return _run(logits, target)
